# baseline (speedup 1.0000x reference)
import jax
import jax.numpy as jnp
import numpy as np
from jax import lax
from jax.experimental import pallas as pl
from jax.experimental.pallas import tpu as pltpu
from jax.experimental.pallas import tpu_sc as plsc

N_ROWS = 1024
VOCAB = 100000
SMOOTH = 0.1
EPS = np.float32(SMOOTH / (VOCAB - 2))
COEF = np.float32((1.0 - SMOOTH) - SMOOTH / (VOCAB - 2))

NC = 2
NS = 16
NW = NC * NS
ROWS_PER_TILE = N_ROWS // NW
CH = VOCAB // 2
VECS = CH // 16
UNROLL = 25


def _chunk_sum(buf):
    zero = jnp.zeros((16,), jnp.float32)
    accs = (zero,) * 5

    def body(i, accs):
        base = i * (16 * UNROLL)
        new = list(accs)
        for j in range(UNROLL):
            v = buf[pl.ds(base + j * 16, 16)]
            new[j % 5] = new[j % 5] + v
        return tuple(new)

    accs = lax.fori_loop(0, VECS // UNROLL, body, accs)
    return ((accs[0] + accs[1]) + (accs[2] + accs[3])) + accs[4]


def _loss_kernel(pred_hbm, tgt_hbm, out_hbm, tgt_v, buf0, buf1, outv,
                 sem0, sem1):
    wid = lax.axis_index("s") * NC + lax.axis_index("c")
    base_row = wid * ROWS_PER_TILE

    def start(row, c, buf, sem):
        pltpu.async_copy(pred_hbm.at[pl.ds(row * VOCAB + c * CH, CH)],
                         buf, sem)

    def wait(buf, sem):
        pltpu.make_async_copy(pred_hbm.at[pl.ds(0, CH)], buf, sem).wait()

    pltpu.sync_copy(tgt_hbm.at[pl.ds(base_row, ROWS_PER_TILE)], tgt_v)
    start(base_row, 0, buf0, sem0)
    start(base_row, 1, buf1, sem1)

    lane = lax.iota(jnp.int32, 16)
    lane0 = lane == 0
    zeros_i = jnp.zeros((16,), jnp.int32)
    fzero = jnp.zeros((16,), jnp.float32)

    def body(r, carry):
        acc_s, acc_g = carry
        row = base_row + r
        t_b = plsc.load_gather(tgt_v, [jnp.full((16,), r, jnp.int32)])
        tmask = t_b != 0

        wait(buf0, sem0)
        sum0 = _chunk_sum(buf0)
        loc0 = jnp.minimum(t_b, CH - 1)
        g0 = plsc.load_gather(buf0, [loc0])
        p0 = plsc.load_gather(buf0, [zeros_i])

        @pl.when(r < ROWS_PER_TILE - 1)
        def _():
            start(row + 1, 0, buf0, sem0)

        wait(buf1, sem1)
        sum1 = _chunk_sum(buf1)
        loc1 = jnp.clip(t_b - CH, 0, CH - 1)
        g1 = plsc.load_gather(buf1, [loc1])

        @pl.when(r < ROWS_PER_TILE - 1)
        def _():
            start(row + 1, 1, buf1, sem1)

        tv = jnp.where(t_b < CH, g0, g1)
        acc_s = acc_s + jnp.where(tmask, sum0 + sum1, fzero)
        acc_g = acc_g + jnp.where(tmask & lane0, COEF * tv - EPS * p0, fzero)
        return acc_s, acc_g

    acc_s, acc_g = lax.fori_loop(0, ROWS_PER_TILE, body, (fzero, fzero))
    outv[...] = (EPS * acc_s + acc_g) * np.float32(-1.0 / N_ROWS)
    pltpu.sync_copy(outv, out_hbm.at[wid])


@jax.jit
def kernel(pred, target):
    mesh = plsc.VectorSubcoreMesh(core_axis_name="c", subcore_axis_name="s")
    partials = pl.kernel(
        _loss_kernel,
        mesh=mesh,
        compiler_params=pltpu.CompilerParams(needs_layout_passes=False),
        out_type=jax.ShapeDtypeStruct((NW, 16), jnp.float32),
        scratch_types=[
            pltpu.VMEM((ROWS_PER_TILE,), jnp.int32),
            pltpu.VMEM((CH,), jnp.float32),
            pltpu.VMEM((CH,), jnp.float32),
            pltpu.VMEM((16,), jnp.float32),
            pltpu.SemaphoreType.DMA,
            pltpu.SemaphoreType.DMA,
        ],
    )(pred.reshape(-1), target)
    return jnp.sum(partials)


# speedup vs baseline: 1.0074x; 1.0074x over previous
import jax
import jax.numpy as jnp
import numpy as np
from jax import lax
from jax.experimental import pallas as pl
from jax.experimental.pallas import tpu as pltpu
from jax.experimental.pallas import tpu_sc as plsc

N_ROWS = 1024
VOCAB = 100000
SMOOTH = 0.1
EPS = np.float32(SMOOTH / (VOCAB - 2))
COEF = np.float32((1.0 - SMOOTH) - SMOOTH / (VOCAB - 2))

NC = 2
NS = 16
NW = NC * NS
ROWS_PER_TILE = N_ROWS // NW
NB = 10
CH = VOCAB // NB
VECS = CH // 16
UNROLL = 25


def _chunk_sum(buf):
    zero = jnp.zeros((16,), jnp.float32)
    accs = (zero,) * 5

    def body(i, accs):
        base = i * (16 * UNROLL)
        new = list(accs)
        for j in range(UNROLL):
            v = buf[pl.ds(base + j * 16, 16)]
            new[j % 5] = new[j % 5] + v
        return tuple(new)

    accs = lax.fori_loop(0, VECS // UNROLL, body, accs)
    return ((accs[0] + accs[1]) + (accs[2] + accs[3])) + accs[4]


def _loss_kernel(pred_hbm, tgt_hbm, out_hbm, tgt_v, outv, *bufs_sems):
    bufs = bufs_sems[:NB]
    sems = bufs_sems[NB:]
    wid = lax.axis_index("s") * NC + lax.axis_index("c")
    base_row = wid * ROWS_PER_TILE

    def start(row, c, b):
        pltpu.async_copy(pred_hbm.at[pl.ds(row * VOCAB + c * CH, CH)],
                         bufs[b], sems[b])

    def wait(b):
        pltpu.make_async_copy(pred_hbm.at[pl.ds(0, CH)], bufs[b],
                              sems[b]).wait()

    pltpu.sync_copy(tgt_hbm.at[pl.ds(base_row, ROWS_PER_TILE)], tgt_v)
    for b in range(NB):
        start(base_row, b, b)

    lane = lax.iota(jnp.int32, 16)
    lane0 = lane == 0
    fzero = jnp.zeros((16,), jnp.float32)

    def body(r, carry):
        acc_s, acc_g = carry
        row = base_row + r
        t_b = plsc.load_gather(tgt_v, [jnp.full((16,), r, jnp.int32)])
        tmask = t_b != 0

        row_acc = fzero
        tv = fzero
        p0 = fzero
        for b in range(NB):
            wait(b)
            row_acc = row_acc + _chunk_sum(bufs[b])
            loc = jnp.clip(t_b - b * CH, 0, CH - 1)
            g = plsc.load_gather(bufs[b], [loc])
            in_b = (t_b >= b * CH) & (t_b < (b + 1) * CH)
            tv = jnp.where(in_b, g, tv)
            if b == 0:
                p0 = plsc.load_gather(bufs[0], [jnp.zeros((16,), jnp.int32)])

            @pl.when(r < ROWS_PER_TILE - 1)
            def _():
                start(row + 1, b, b)

        acc_s = acc_s + jnp.where(tmask, row_acc, fzero)
        acc_g = acc_g + jnp.where(tmask & lane0, COEF * tv - EPS * p0, fzero)
        return acc_s, acc_g

    acc_s, acc_g = lax.fori_loop(0, ROWS_PER_TILE, body, (fzero, fzero))
    outv[...] = (EPS * acc_s + acc_g) * np.float32(-1.0 / N_ROWS)
    pltpu.sync_copy(outv, out_hbm.at[wid])


@jax.jit
def kernel(pred, target):
    mesh = plsc.VectorSubcoreMesh(core_axis_name="c", subcore_axis_name="s")
    partials = pl.kernel(
        _loss_kernel,
        mesh=mesh,
        compiler_params=pltpu.CompilerParams(needs_layout_passes=False),
        out_type=jax.ShapeDtypeStruct((NW, 16), jnp.float32),
        scratch_types=(
            [pltpu.VMEM((ROWS_PER_TILE,), jnp.int32),
             pltpu.VMEM((16,), jnp.float32)]
            + [pltpu.VMEM((CH,), jnp.float32) for _ in range(NB)]
            + [pltpu.SemaphoreType.DMA for _ in range(NB)]
        ),
    )(pred.reshape(-1), target)
    return jnp.sum(partials)
